# baseline (speedup 1.0000x reference)
import functools

import jax
import jax.numpy as jnp
from jax import lax
from jax.experimental import pallas as pl
from jax.experimental.pallas import tpu as pltpu
from jax.experimental.pallas import tpu_sc as plsc

N = 10000
D = 128
E = 320000
NC, NS, L = 2, 16, 16
NW = NC * NS
CH = 80
EPAD = 327680
NCHT = EPAD // CH
EPT1 = EPAD // NW
EPT2 = EPAD // NS
NP = 10112
RPT = NP // NS
NDEG = 10240
NBUF = 4
GLA = 3

_MESH = dict(core_axis_name="c", subcore_axis_name="s", num_cores=NC,
             num_subcores=NS)


def _deg_body(dj_hbm, ew_hbm, z_hbm, out_hbm, *scr):
    jb = scr[0:2]
    wb = scr[2:4]
    dc = scr[4:6]
    wc = scr[6:8]
    acc = scr[8]
    se = scr[9:11]
    ss = scr[11:13]
    c = lax.axis_index("c")
    s = lax.axis_index("s")
    wid = c * NS + s
    seg = NDEG // NS
    pltpu.sync_copy(z_hbm.at[pl.ds(s * seg, seg)], acc.at[pl.ds(s * seg, seg)])
    plsc.subcore_barrier()

    nch = EPT1 // CH
    cbase = wid * nch

    def start_e(k, b):
        sl = pl.ds((cbase + k) * CH, CH)
        pltpu.async_copy(dj_hbm.at[sl], jb[b], se[b])
        pltpu.async_copy(ew_hbm.at[sl], wb[b], se[b])

    def wait_e(b):
        sl = pl.ds(0, CH)
        pltpu.make_async_copy(dj_hbm.at[sl], jb[b], se[b]).wait()
        pltpu.make_async_copy(ew_hbm.at[sl], wb[b], se[b]).wait()

    def wait_s(b):
        pltpu.make_async_copy(wc[b], acc.at[dc[b]], ss[b]).wait()

    start_e(0, 0)
    start_e(1, 1)

    def step(i, carry):
        for b in range(2):
            @pl.when(lax.rem(i, 2) == b)
            def _():
                wait_e(b)
                for g in range(CH // L):
                    sl = pl.ds(g * L, L)
                    dc[b][sl] = jb[b][sl]
                    wc[b][sl] = wb[b][sl]

                @pl.when(i >= 2)
                def _():
                    wait_s(b)

                pltpu.async_copy(wc[b], acc.at[dc[b]], ss[b], add=True)

                @pl.when(i + 2 < nch)
                def _():
                    start_e(i + 2, b)

        return carry

    lax.fori_loop(0, nch, step, 0)
    wait_s(nch % 2)
    wait_s((nch + 1) % 2)
    plsc.subcore_barrier()
    pltpu.sync_copy(acc.at[pl.ds(s * seg, seg)],
                    out_hbm.at[c, 0, pl.ds(s * seg, seg)])


_deg = pl.kernel(
    _deg_body,
    out_type=jax.ShapeDtypeStruct((NC, 1, NDEG), jnp.float32),
    mesh=plsc.VectorSubcoreMesh(**_MESH),
    scratch_types=(
        [pltpu.VMEM((CH,), jnp.int32)] * 2
        + [pltpu.VMEM((CH,), jnp.float32)] * 2
        + [pltpu.VMEM((CH,), jnp.int32)] * 2
        + [pltpu.VMEM((CH,), jnp.float32)] * 2
        + [pltpu.VMEM_SHARED((NDEG,), jnp.float32)]
        + [pltpu.SemaphoreType.DMA] * 4
    ),
)


def _prop_body(nplanes, g_hbm, sa_hbm, sb_hbm, dj_hbm, ew_hbm, z_hbm, out_hbm,
               *scr):
    nb = NBUF
    ib = scr[0 : nb]
    jb = scr[nb : 2 * nb]
    wb = scr[2 * nb : 3 * nb]
    dc = scr[3 * nb : 4 * nb]
    rw = scr[4 * nb : 5 * nb]
    acc = scr[5 * nb]
    se = scr[5 * nb + 1 : 6 * nb + 1]
    sg = scr[6 * nb + 1 : 7 * nb + 1]
    ss = scr[7 * nb + 1 : 8 * nb + 1]
    c = lax.axis_index("c")
    s = lax.axis_index("s")
    myrow = s * RPT
    pltpu.sync_copy(z_hbm.at[pl.ds(myrow, RPT)], acc.at[pl.ds(myrow, RPT)])
    plsc.subcore_barrier()

    if nplanes == 1:
        cbase = (c * NS + s) * (EPT1 // CH)
        nch = EPT1 // CH
    else:
        cbase = s * (EPT2 // CH)
        nch = EPT2 // CH

    def start_e(k, b):
        sl = pl.ds((cbase + k) * CH, CH)
        if nplanes == 1:
            pltpu.async_copy(sa_hbm.at[sl], ib[b], se[b])
        else:
            @pl.when(c == 0)
            def _():
                pltpu.async_copy(sa_hbm.at[sl], ib[b], se[b])

            @pl.when(c == 1)
            def _():
                pltpu.async_copy(sb_hbm.at[sl], ib[b], se[b])

        pltpu.async_copy(dj_hbm.at[sl], jb[b], se[b])
        pltpu.async_copy(ew_hbm.at[sl], wb[b], se[b])

    def wait_e(b):
        sl = pl.ds(0, CH)
        pltpu.make_async_copy(sa_hbm.at[sl], ib[b], se[b]).wait()
        pltpu.make_async_copy(dj_hbm.at[sl], jb[b], se[b]).wait()
        pltpu.make_async_copy(ew_hbm.at[sl], wb[b], se[b]).wait()

    def start_g(b):
        pltpu.async_copy(g_hbm.at[ib[b]], rw[b], sg[b])

    def wait_g(b):
        pltpu.make_async_copy(g_hbm.at[ib[b]], rw[b], sg[b]).wait()

    def wait_s(b):
        pltpu.make_async_copy(rw[b], acc.at[dc[b]], ss[b]).wait()

    for k in range(NBUF):
        start_e(k, k)
    for k in range(GLA):
        wait_e(k)
        start_g(k)

    def step(i, carry):
        for b in range(NBUF):
            @pl.when(lax.rem(i, NBUF) == b)
            def _():
                b2 = (b + GLA) % NBUF
                wait_g(b)

                @pl.when(i + GLA < nch)
                def _():
                    @pl.when(i >= GLA)
                    def _():
                        wait_s(b2)

                    wait_e(b2)
                    start_g(b2)

                def group(g, inner):
                    wg = wb[b][pl.ds(g * L, L)]
                    dc[b][pl.ds(g * L, L)] = jb[b][pl.ds(g * L, L)]
                    for l in range(L):
                        wv = jnp.broadcast_to(wg[l], (L,))
                        e = g * L + l
                        for j in range(D // L):
                            sl = pl.ds(j * L, L)
                            rw[b][e, sl] = rw[b][e, sl] * wv
                    return inner

                lax.fori_loop(0, CH // L, group, 0)
                pltpu.async_copy(rw[b], acc.at[dc[b]], ss[b], add=True)

                @pl.when(i + NBUF < nch)
                def _():
                    start_e(i + NBUF, b)

        return carry

    lax.fori_loop(0, nch, step, 0)
    for k in range(NBUF):
        wait_s((nch - NBUF + k) % NBUF)
    plsc.subcore_barrier()
    pltpu.sync_copy(acc.at[pl.ds(myrow, RPT)], out_hbm.at[c, pl.ds(myrow, RPT)])


def _make_prop(nplanes):
    return pl.kernel(
        functools.partial(_prop_body, nplanes),
        out_type=jax.ShapeDtypeStruct((NC, NP, D), jnp.float32),
        mesh=plsc.VectorSubcoreMesh(**_MESH),
        scratch_types=(
            [pltpu.VMEM((CH,), jnp.int32)] * NBUF
            + [pltpu.VMEM((CH,), jnp.int32)] * NBUF
            + [pltpu.VMEM((CH,), jnp.float32)] * NBUF
            + [pltpu.VMEM((CH,), jnp.int32)] * NBUF
            + [pltpu.VMEM((CH, D), jnp.float32)] * NBUF
            + [pltpu.VMEM_SHARED((NP, D), jnp.float32)]
            + [pltpu.SemaphoreType.DMA] * (3 * NBUF)
        ),
    )


_prop128 = _make_prop(1)
_prop256 = _make_prop(2)


def _dot(a, b):
    return jnp.dot(a, b, preferred_element_type=jnp.float32,
                   precision=lax.Precision.HIGHEST)


def _tc1_body(deg_ref, x_ref, w1_ref, g1_ref, dinv_ref):
    deg = deg_ref[0, 0, :] + deg_ref[1, 0, :] + 1.0
    dinv1 = lax.rsqrt(deg)
    dinvb = jnp.broadcast_to(dinv1[:N][:, None], (N, D))
    dinv_ref[...] = dinvb
    g1_ref[...] = dinvb * _dot(x_ref[...], w1_ref[...])


def _tc2_body(p_ref, g_ref, dinv_ref, b_ref, out_ref):
    d = dinv_ref[...]
    x1 = jnp.maximum(d * (p_ref[0, :N] + p_ref[1, :N] + g_ref[...])
                     + b_ref[...], 0.0)
    out_ref[...] = d * x1


def _tc3_body(p_ref, g_ref, dinv_ref, w2_ref, b2_ref, w3_ref, g3_ref):
    d = dinv_ref[...]
    q = d * (p_ref[0] + p_ref[1] + g_ref[...])
    x2 = jnp.maximum(_dot(q, w2_ref[...]) + b2_ref[...], 0.0)
    h3 = _dot(x2, w3_ref[...])
    g3_ref[0] = d * h3[:, :D]
    g3_ref[1] = d * h3[:, D:]


def _tc4_body(p_ref, g_ref, dinv_ref, b3_ref, w4_ref, out_ref):
    d = dinv_ref[...]
    x3a = jnp.maximum(d * (p_ref[0] + g_ref[0]) + b3_ref[:, :D], 0.0)
    x3b = jnp.maximum(d * (p_ref[1] + g_ref[1]) + b3_ref[:, D:], 0.0)
    x3 = jnp.concatenate([x3a, x3b], axis=1)
    out_ref[...] = d * _dot(x3, w4_ref[...])


_R = 1000


def _rowspec(nrows):
    return pl.BlockSpec((_R, nrows), lambda i: (i, 0))


def _planespec():
    return pl.BlockSpec((NC, _R, D), lambda i: (0, i, 0))


def _fullspec(shape):
    return pl.BlockSpec(shape, lambda i: tuple(0 for _ in shape))


def _tc_grid(body, out_shapes, out_specs, args, in_specs):
    outs = [jax.ShapeDtypeStruct(s, jnp.float32) for s in out_shapes]
    return pl.pallas_call(body, out_shape=outs, grid=(N // _R,),
                          in_specs=in_specs, out_specs=out_specs)(*args)


def _tc5_body(p_ref, g_ref, dinv_ref, b4_ref, w5_ref, out_ref):
    d = dinv_ref[...]
    x4 = jnp.maximum(d * (p_ref[0, :N] + p_ref[1, :N] + g_ref[...])
                     + b4_ref[...], 0.0)
    out_ref[...] = d * _dot(x4, w5_ref[...])


def _tc6_body(p_ref, g_ref, dinv_ref, b5_ref, out_ref):
    out_ref[...] = (dinv_ref[...] * (p_ref[0, :N] + p_ref[1, :N] + g_ref[...])
                    + b5_ref[...])


def _tc(body, out_shapes, *args):
    outs = [jax.ShapeDtypeStruct(s, jnp.float32) for s in out_shapes]
    res = pl.pallas_call(body, out_shape=outs)(*args)
    return res


def kernel(edge_index, edge_weight, node_emb, W1, b1, W2, b2, W3, b3, W4, b4,
           W5, b5):
    src = edge_index[0].astype(jnp.int32)
    dst = edge_index[1].astype(jnp.int32)
    w = edge_weight.astype(jnp.float32)
    pad = EPAD - E
    pad_src = jnp.arange(pad, dtype=jnp.int32) % N
    pad_dst = N + jnp.arange(pad, dtype=jnp.int32) % (NP - N)
    src_p = jnp.concatenate([src, pad_src])
    dst_p = jnp.concatenate([dst, pad_dst])
    w_p = jnp.concatenate([w, jnp.zeros((pad,), jnp.float32)])
    sa = src_p
    sb = src_p + N
    dj = dst_p
    ew = w_p
    z2d = jnp.zeros((NP, D), jnp.float32)
    z1d = jnp.zeros((NDEG,), jnp.float32)
    b1r, b2r, b3r, b4r, b5r = (b.reshape(1, -1) for b in (b1, b2, b3, b4, b5))

    deg = _deg(dj, ew, z1d)
    g1, dinvb = _tc(_tc1_body, [(N, D), (N, D)], deg, node_emb, W1)
    p1 = _prop128(g1, sa, sb, dj, ew, z2d)
    (g2,) = _tc(_tc2_body, [(N, D)], p1, g1, dinvb, b1r)
    p2 = _prop128(g2, sa, sb, dj, ew, z2d)
    (g3,) = _tc_grid(
        _tc3_body, [(NC, N, D)], [_planespec()],
        (p2, g2, dinvb, W2, b2r, W3),
        [_planespec(), _rowspec(D), _rowspec(D), _fullspec((D, 2 * D)),
         _fullspec((1, 2 * D)), _fullspec((2 * D, 2 * D))])
    p3 = _prop256(g3.reshape(NC * N, D), sa, sb, dj, ew, z2d)
    (g4,) = _tc_grid(
        _tc4_body, [(N, D)], [_rowspec(D)],
        (p3, g3, dinvb, b3r, W4),
        [_planespec(), _planespec(), _rowspec(D), _fullspec((1, 2 * D)),
         _fullspec((2 * D, D))])
    p4 = _prop128(g4, sa, sb, dj, ew, z2d)
    (g5,) = _tc(_tc5_body, [(N, D)], p4, g4, dinvb, b4r, W5)
    p5 = _prop128(g5, sa, sb, dj, ew, z2d)
    (out,) = _tc(_tc6_body, [(N, D)], p5, g5, dinvb, b5r)
    return out


# speedup vs baseline: 1.1807x; 1.1807x over previous
import functools

import jax
import jax.numpy as jnp
from jax import lax
from jax.experimental import pallas as pl
from jax.experimental.pallas import tpu as pltpu
from jax.experimental.pallas import tpu_sc as plsc

N = 10000
D = 128
E = 320000
NC, NS, L = 2, 16, 16
NW = NC * NS
CH = 80
EPAD = 327680
NCHT = EPAD // CH
EPT1 = EPAD // NW
EPT2 = EPAD // NS
NP = 10112
RPT = NP // NS
NDEG = 10240
NBUF = 4
GLA = 2

_MESH = dict(core_axis_name="c", subcore_axis_name="s", num_cores=NC,
             num_subcores=NS)


def _deg_body(dj_hbm, ew_hbm, z_hbm, out_hbm, *scr):
    jb = scr[0:2]
    wb = scr[2:4]
    dc = scr[4:6]
    wc = scr[6:8]
    acc = scr[8]
    se = scr[9:11]
    ss = scr[11:13]
    c = lax.axis_index("c")
    s = lax.axis_index("s")
    wid = c * NS + s
    seg = NDEG // NS
    pltpu.sync_copy(z_hbm.at[pl.ds(s * seg, seg)], acc.at[pl.ds(s * seg, seg)])
    plsc.subcore_barrier()

    nch = EPT1 // CH
    cbase = wid * nch

    def start_e(k, b):
        sl = pl.ds((cbase + k) * CH, CH)
        pltpu.async_copy(dj_hbm.at[sl], jb[b], se[b])
        pltpu.async_copy(ew_hbm.at[sl], wb[b], se[b])

    def wait_e(b):
        sl = pl.ds(0, CH)
        pltpu.make_async_copy(dj_hbm.at[sl], jb[b], se[b]).wait()
        pltpu.make_async_copy(ew_hbm.at[sl], wb[b], se[b]).wait()

    def wait_s(b):
        pltpu.make_async_copy(wc[b], acc.at[dc[b]], ss[b]).wait()

    start_e(0, 0)
    start_e(1, 1)

    def step(i, carry):
        for b in range(2):
            @pl.when(lax.rem(i, 2) == b)
            def _():
                wait_e(b)
                for g in range(CH // L):
                    sl = pl.ds(g * L, L)
                    dc[b][sl] = jb[b][sl]
                    wc[b][sl] = wb[b][sl]

                @pl.when(i >= 2)
                def _():
                    wait_s(b)

                pltpu.async_copy(wc[b], acc.at[dc[b]], ss[b], add=True)

                @pl.when(i + 2 < nch)
                def _():
                    start_e(i + 2, b)

        return carry

    lax.fori_loop(0, nch, step, 0)
    wait_s(nch % 2)
    wait_s((nch + 1) % 2)
    plsc.subcore_barrier()
    pltpu.sync_copy(acc.at[pl.ds(s * seg, seg)],
                    out_hbm.at[c, 0, pl.ds(s * seg, seg)])


_deg = pl.kernel(
    _deg_body,
    out_type=jax.ShapeDtypeStruct((NC, 1, NDEG), jnp.float32),
    mesh=plsc.VectorSubcoreMesh(**_MESH),
    scratch_types=(
        [pltpu.VMEM((CH,), jnp.int32)] * 2
        + [pltpu.VMEM((CH,), jnp.float32)] * 2
        + [pltpu.VMEM((CH,), jnp.int32)] * 2
        + [pltpu.VMEM((CH,), jnp.float32)] * 2
        + [pltpu.VMEM_SHARED((NDEG,), jnp.float32)]
        + [pltpu.SemaphoreType.DMA] * 4
    ),
)


def _prop_body(nplanes, g_hbm, sa_hbm, sb_hbm, dj_hbm, ew_hbm, z_hbm, out_hbm,
               *scr):
    nb = NBUF
    ib = scr[0 : nb]
    jb = scr[nb : 2 * nb]
    wb = scr[2 * nb : 3 * nb]
    dc = scr[3 * nb : 4 * nb]
    rw = scr[4 * nb : 5 * nb]
    acc = scr[5 * nb]
    se = scr[5 * nb + 1 : 6 * nb + 1]
    sg = scr[6 * nb + 1 : 7 * nb + 1]
    ss = scr[7 * nb + 1 : 8 * nb + 1]
    c = lax.axis_index("c")
    s = lax.axis_index("s")
    myrow = s * RPT
    pltpu.sync_copy(z_hbm.at[pl.ds(myrow, RPT)], acc.at[pl.ds(myrow, RPT)])
    plsc.subcore_barrier()

    if nplanes == 1:
        cbase = (c * NS + s) * (EPT1 // CH)
        nch = EPT1 // CH
    else:
        cbase = s * (EPT2 // CH)
        nch = EPT2 // CH

    def start_e(k, b):
        sl = pl.ds((cbase + k) * CH, CH)
        if nplanes == 1:
            pltpu.async_copy(sa_hbm.at[sl], ib[b], se[b])
        else:
            @pl.when(c == 0)
            def _():
                pltpu.async_copy(sa_hbm.at[sl], ib[b], se[b])

            @pl.when(c == 1)
            def _():
                pltpu.async_copy(sb_hbm.at[sl], ib[b], se[b])

        pltpu.async_copy(dj_hbm.at[sl], jb[b], se[b])
        pltpu.async_copy(ew_hbm.at[sl], wb[b], se[b])

    def wait_e(b):
        sl = pl.ds(0, CH)
        pltpu.make_async_copy(sa_hbm.at[sl], ib[b], se[b]).wait()
        pltpu.make_async_copy(dj_hbm.at[sl], jb[b], se[b]).wait()
        pltpu.make_async_copy(ew_hbm.at[sl], wb[b], se[b]).wait()

    def start_g(b):
        pltpu.async_copy(g_hbm.at[ib[b]], rw[b], sg[b])

    def wait_g(b):
        pltpu.make_async_copy(g_hbm.at[ib[b]], rw[b], sg[b]).wait()

    def wait_s(b):
        pltpu.make_async_copy(rw[b], acc.at[dc[b]], ss[b]).wait()

    for k in range(NBUF):
        start_e(k, k)
    for k in range(GLA):
        wait_e(k)
        start_g(k)

    def step(i, carry):
        for b in range(NBUF):
            @pl.when(lax.rem(i, NBUF) == b)
            def _():
                b2 = (b + GLA) % NBUF
                wait_g(b)

                @pl.when(i + GLA < nch)
                def _():
                    @pl.when(i >= GLA)
                    def _():
                        wait_s(b2)

                    wait_e(b2)
                    start_g(b2)

                def group(g, inner):
                    wg = wb[b][pl.ds(g * L, L)]
                    dc[b][pl.ds(g * L, L)] = jb[b][pl.ds(g * L, L)]
                    for l in range(L):
                        wv = jnp.broadcast_to(wg[l], (L,))
                        e = g * L + l
                        for j in range(D // L):
                            sl = pl.ds(j * L, L)
                            rw[b][e, sl] = rw[b][e, sl] * wv
                    return inner

                lax.fori_loop(0, CH // L, group, 0)
                pltpu.async_copy(rw[b], acc.at[dc[b]], ss[b], add=True)

                @pl.when(i + NBUF < nch)
                def _():
                    start_e(i + NBUF, b)

        return carry

    lax.fori_loop(0, nch, step, 0)
    for k in range(NBUF):
        wait_s((nch - NBUF + k) % NBUF)
    plsc.subcore_barrier()
    pltpu.sync_copy(acc.at[pl.ds(myrow, RPT)], out_hbm.at[c, pl.ds(myrow, RPT)])


def _make_prop(nplanes):
    return pl.kernel(
        functools.partial(_prop_body, nplanes),
        out_type=jax.ShapeDtypeStruct((NC, NP, D), jnp.float32),
        mesh=plsc.VectorSubcoreMesh(**_MESH),
        scratch_types=(
            [pltpu.VMEM((CH,), jnp.int32)] * NBUF
            + [pltpu.VMEM((CH,), jnp.int32)] * NBUF
            + [pltpu.VMEM((CH,), jnp.float32)] * NBUF
            + [pltpu.VMEM((CH,), jnp.int32)] * NBUF
            + [pltpu.VMEM((CH, D), jnp.float32)] * NBUF
            + [pltpu.VMEM_SHARED((NP, D), jnp.float32)]
            + [pltpu.SemaphoreType.DMA] * (3 * NBUF)
        ),
    )


_prop128 = _make_prop(1)
_prop256 = _make_prop(2)


def _dot(a, b):
    return jnp.dot(a, b, preferred_element_type=jnp.float32,
                   precision=lax.Precision.DEFAULT)


def _tc1_body(deg_ref, x_ref, w1_ref, g1_ref, dinv_ref):
    deg = deg_ref[0, 0, :] + deg_ref[1, 0, :] + 1.0
    dinv1 = lax.rsqrt(deg)
    dinvb = jnp.broadcast_to(dinv1[:N][:, None], (N, D))
    dinv_ref[...] = dinvb
    g1_ref[...] = dinvb * _dot(x_ref[...], w1_ref[...])


def _tc2_body(p_ref, g_ref, dinv_ref, b_ref, out_ref):
    d = dinv_ref[...]
    x1 = jnp.maximum(d * (p_ref[0, :N] + p_ref[1, :N] + g_ref[...])
                     + b_ref[...], 0.0)
    out_ref[...] = d * x1


def _tc3_body(p_ref, g_ref, dinv_ref, w2_ref, b2_ref, w3_ref, g3_ref):
    d = dinv_ref[...]
    q = d * (p_ref[0] + p_ref[1] + g_ref[...])
    x2 = jnp.maximum(_dot(q, w2_ref[...]) + b2_ref[...], 0.0)
    h3 = _dot(x2, w3_ref[...])
    g3_ref[0] = d * h3[:, :D]
    g3_ref[1] = d * h3[:, D:]


def _tc4_body(p_ref, g_ref, dinv_ref, b3_ref, w4_ref, out_ref):
    d = dinv_ref[...]
    x3a = jnp.maximum(d * (p_ref[0] + g_ref[0]) + b3_ref[:, :D], 0.0)
    x3b = jnp.maximum(d * (p_ref[1] + g_ref[1]) + b3_ref[:, D:], 0.0)
    x3 = jnp.concatenate([x3a, x3b], axis=1)
    out_ref[...] = d * _dot(x3, w4_ref[...])


_R = 1000


def _rowspec(nrows):
    return pl.BlockSpec((_R, nrows), lambda i: (i, 0))


def _planespec():
    return pl.BlockSpec((NC, _R, D), lambda i: (0, i, 0))


def _fullspec(shape):
    return pl.BlockSpec(shape, lambda i: tuple(0 for _ in shape))


def _tc_grid(body, out_shapes, out_specs, args, in_specs):
    outs = [jax.ShapeDtypeStruct(s, jnp.float32) for s in out_shapes]
    return pl.pallas_call(body, out_shape=outs, grid=(N // _R,),
                          in_specs=in_specs, out_specs=out_specs)(*args)


def _tc5_body(p_ref, g_ref, dinv_ref, b4_ref, w5_ref, out_ref):
    d = dinv_ref[...]
    x4 = jnp.maximum(d * (p_ref[0, :N] + p_ref[1, :N] + g_ref[...])
                     + b4_ref[...], 0.0)
    out_ref[...] = d * _dot(x4, w5_ref[...])


def _tc6_body(p_ref, g_ref, dinv_ref, b5_ref, out_ref):
    out_ref[...] = (dinv_ref[...] * (p_ref[0, :N] + p_ref[1, :N] + g_ref[...])
                    + b5_ref[...])


def _tc(body, out_shapes, *args):
    outs = [jax.ShapeDtypeStruct(s, jnp.float32) for s in out_shapes]
    res = pl.pallas_call(body, out_shape=outs)(*args)
    return res


def kernel(edge_index, edge_weight, node_emb, W1, b1, W2, b2, W3, b3, W4, b4,
           W5, b5):
    src = edge_index[0].astype(jnp.int32)
    dst = edge_index[1].astype(jnp.int32)
    w = edge_weight.astype(jnp.float32)
    pad = EPAD - E
    pad_src = jnp.arange(pad, dtype=jnp.int32) % N
    pad_dst = N + jnp.arange(pad, dtype=jnp.int32) % (NP - N)
    src_p = jnp.concatenate([src, pad_src])
    dst_p = jnp.concatenate([dst, pad_dst])
    w_p = jnp.concatenate([w, jnp.zeros((pad,), jnp.float32)])
    sa = src_p
    sb = src_p + N
    dj = dst_p
    ew = w_p
    z2d = jnp.zeros((NP, D), jnp.float32)
    z1d = jnp.zeros((NDEG,), jnp.float32)
    b1r, b2r, b3r, b4r, b5r = (b.reshape(1, -1) for b in (b1, b2, b3, b4, b5))

    deg = _deg(dj, ew, z1d)
    g1, dinvb = _tc(_tc1_body, [(N, D), (N, D)], deg, node_emb, W1)
    p1 = _prop128(g1, sa, sb, dj, ew, z2d)
    (g2,) = _tc(_tc2_body, [(N, D)], p1, g1, dinvb, b1r)
    p2 = _prop128(g2, sa, sb, dj, ew, z2d)
    (g3,) = _tc_grid(
        _tc3_body, [(NC, N, D)], [_planespec()],
        (p2, g2, dinvb, W2, b2r, W3),
        [_planespec(), _rowspec(D), _rowspec(D), _fullspec((D, 2 * D)),
         _fullspec((1, 2 * D)), _fullspec((2 * D, 2 * D))])
    p3 = _prop256(g3.reshape(NC * N, D), sa, sb, dj, ew, z2d)
    (g4,) = _tc_grid(
        _tc4_body, [(N, D)], [_rowspec(D)],
        (p3, g3, dinvb, b3r, W4),
        [_planespec(), _planespec(), _rowspec(D), _fullspec((1, 2 * D)),
         _fullspec((2 * D, D))])
    p4 = _prop128(g4, sa, sb, dj, ew, z2d)
    (g5,) = _tc(_tc5_body, [(N, D)], p4, g4, dinvb, b4r, W5)
    p5 = _prop128(g5, sa, sb, dj, ew, z2d)
    (out,) = _tc(_tc6_body, [(N, D)], p5, g5, dinvb, b5r)
    return out
